# baseline (speedup 1.0000x reference)
import functools
import math

import jax
import jax.numpy as jnp
from jax import lax
from jax.experimental import pallas as pl
from jax.experimental.pallas import tpu as pltpu
from jax.experimental.pallas import tpu_sc as plsc

_NC = 2
_NS = 16
_NW = _NC * _NS

_CHUNK = 128
_GB = 8


def _gather_body(table_hbm, idx_hbm, out_hbm, idx_v, rows_v, sem):
    wid = lax.axis_index("s") * _NC + lax.axis_index("c")
    n_chunks = idx_hbm.shape[0]
    per_w = n_chunks // _NW

    def step(i, carry):
        base = wid * per_w + i * _GB
        pltpu.sync_copy(idx_hbm.at[pl.ds(base, _GB)], idx_v)
        copies = [
            pltpu.async_copy(table_hbm.at[idx_v.at[j]], rows_v.at[j], sem)
            for j in range(_GB)
        ]
        for c in copies:
            c.wait()
        pltpu.sync_copy(rows_v, out_hbm.at[pl.ds(base, _GB)])
        return carry

    lax.fori_loop(0, per_w // _GB, step, 0)


def _make_gather(n_chunks, embed):
    mesh = plsc.VectorSubcoreMesh(
        core_axis_name="c", subcore_axis_name="s",
        num_cores=_NC, num_subcores=_NS,
    )
    return pl.kernel(
        _gather_body,
        out_type=jax.ShapeDtypeStruct((n_chunks, _CHUNK, embed), jnp.float32),
        mesh=mesh,
        scratch_types=[
            pltpu.VMEM((_GB, _CHUNK), jnp.int32),
            pltpu.VMEM((_GB, _CHUNK, embed), jnp.float32),
            pltpu.SemaphoreType.DMA,
        ],
        compiler_params=pltpu.CompilerParams(use_tc_tiling_on_sc=False),
    )


def _pack_body(a_ref, b_ref, c_ref, d_ref, out_ref):
    out_ref[...] = jnp.concatenate(
        [a_ref[...], b_ref[...], c_ref[...], d_ref[...]], axis=1
    )


def _pack_table(table):
    v, e = table.shape
    q = v // 4
    blk = 2000
    specs = [
        pl.BlockSpec((blk, e), functools.partial(lambda k, i: (i + k * (q // blk), 0), k))
        for k in range(4)
    ]
    return pl.pallas_call(
        _pack_body,
        grid=(q // blk,),
        in_specs=specs,
        out_specs=pl.BlockSpec((blk, 4 * e), lambda i: (i, 0)),
        out_shape=jax.ShapeDtypeStruct((q, 4 * e), jnp.float32),
        compiler_params=pltpu.CompilerParams(
            dimension_semantics=("arbitrary",),
        ),
    )(table, table, table, table)


def _proj_body(e_ref, wt_ref, b_ref, out_ref):
    e = e_ref[...]
    parts = []
    for k in range(4):
        ek = e[:, 32 * k:32 * (k + 1)]
        ok = lax.dot_general(
            wt_ref[...], ek,
            dimension_numbers=(((1,), (1,)), ((), ())),
            preferred_element_type=jnp.float32,
        )
        parts.append(ok)
    out_ref[0] = jnp.concatenate(parts, axis=1) + b_ref[...]


def _proj(emb_pk, wt, b8, hist, batch):
    rows = emb_pk.shape[0]
    rows_per_l = rows // hist
    d = wt.shape[0]
    return pl.pallas_call(
        _proj_body,
        grid=(hist,),
        in_specs=[
            pl.BlockSpec((rows_per_l, 128), lambda i: (i, 0)),
            pl.BlockSpec((d, 32), lambda i: (0, 0)),
            pl.BlockSpec((d, 1), lambda i: (0, 0)),
        ],
        out_specs=pl.BlockSpec((1, d, batch), lambda i: (i, 0, 0)),
        out_shape=jax.ShapeDtypeStruct((hist, d, batch), jnp.float32),
        compiler_params=pltpu.CompilerParams(
            dimension_semantics=("arbitrary",),
        ),
    )(emb_pk, wt, b8)


def kernel(x, table, W, b):
    batch, hist = x.shape
    n = batch * hist
    embed = table.shape[1]
    d_model = W.shape[1]
    scale = math.sqrt(float(d_model))
    quarter = batch // 4

    vocab = table.shape[0]
    vq = vocab // 4
    table_pk = _pack_table(table)
    table_lin = table_pk.reshape(vocab, embed)

    xi = x.astype(jnp.int32)
    xperm = (xi % vq) * 4 + xi // vq
    xp = xperm.T.reshape(hist, 4, quarter).transpose(0, 2, 1)
    idx = xp.reshape(n // _CHUNK, _CHUNK)
    emb = _make_gather(n // _CHUNK, embed)(table_lin, idx)

    emb_pk = emb.reshape(n // 4, 128)

    wt = (W * scale).T
    b8 = (b * scale).reshape(d_model, 1)
    out3 = _proj(emb_pk, wt, b8, hist, batch)
    return jnp.transpose(out3, (2, 0, 1))


# speedup vs baseline: 1.2372x; 1.2372x over previous
import functools
import math

import jax
import jax.numpy as jnp
from jax import lax
from jax.experimental import pallas as pl
from jax.experimental.pallas import tpu as pltpu
from jax.experimental.pallas import tpu_sc as plsc

_NC = 2
_NS = 16
_NW = _NC * _NS

_CHUNK = 128
_GB = 8


def _gather_body(table_hbm, idx_hbm, out_hbm, idx_v, rows_v, sem):
    wid = lax.axis_index("s") * _NC + lax.axis_index("c")
    n_chunks = idx_hbm.shape[0]
    per_w = n_chunks // _NW

    def step(i, carry):
        base = wid * per_w + i * _GB
        pltpu.sync_copy(idx_hbm.at[pl.ds(base, _GB)], idx_v)
        copies = [
            pltpu.async_copy(table_hbm.at[idx_v.at[j]], rows_v.at[j], sem)
            for j in range(_GB)
        ]
        for c in copies:
            c.wait()
        pltpu.sync_copy(rows_v, out_hbm.at[pl.ds(base, _GB)])
        return carry

    lax.fori_loop(0, per_w // _GB, step, 0)


def _make_gather(n_chunks, embed):
    mesh = plsc.VectorSubcoreMesh(
        core_axis_name="c", subcore_axis_name="s",
        num_cores=_NC, num_subcores=_NS,
    )
    return pl.kernel(
        _gather_body,
        out_type=jax.ShapeDtypeStruct((n_chunks, _CHUNK, embed), jnp.float32),
        mesh=mesh,
        scratch_types=[
            pltpu.VMEM((_GB, _CHUNK), jnp.int32),
            pltpu.VMEM((_GB, _CHUNK, embed), jnp.float32),
            pltpu.SemaphoreType.DMA,
        ],
        compiler_params=pltpu.CompilerParams(use_tc_tiling_on_sc=False),
    )


_LB = 2


def _proj_body(e_ref, wt_ref, b_ref, out_ref):
    rows_per_l = e_ref.shape[0] // _LB
    for m in range(_LB):
        e = e_ref[pl.ds(m * rows_per_l, rows_per_l), :]
        parts = []
        for k in range(4):
            ek = e[:, 32 * k:32 * (k + 1)]
            ok = lax.dot_general(
                wt_ref[...], ek,
                dimension_numbers=(((1,), (1,)), ((), ())),
                preferred_element_type=jnp.float32,
            )
            parts.append(ok)
        out_ref[m] = jnp.concatenate(parts, axis=1) + b_ref[...]


def _proj(emb_pk, wt, b8, hist, batch):
    rows = emb_pk.shape[0]
    rows_per_l = rows // hist
    d = wt.shape[0]
    return pl.pallas_call(
        _proj_body,
        grid=(hist // _LB,),
        in_specs=[
            pl.BlockSpec((_LB * rows_per_l, 128), lambda i: (i, 0)),
            pl.BlockSpec((d, 32), lambda i: (0, 0)),
            pl.BlockSpec((d, 1), lambda i: (0, 0)),
        ],
        out_specs=pl.BlockSpec((_LB, d, batch), lambda i: (i, 0, 0)),
        out_shape=jax.ShapeDtypeStruct((hist, d, batch), jnp.float32),
        compiler_params=pltpu.CompilerParams(
            dimension_semantics=("arbitrary",),
        ),
    )(emb_pk, wt, b8)


def kernel(x, table, W, b):
    batch, hist = x.shape
    n = batch * hist
    embed = table.shape[1]
    d_model = W.shape[1]
    scale = math.sqrt(float(d_model))
    quarter = batch // 4

    xp = x.T.reshape(hist, 4, quarter).transpose(0, 2, 1)
    idx = xp.reshape(n // _CHUNK, _CHUNK).astype(jnp.int32)
    emb = _make_gather(n // _CHUNK, embed)(table, idx)

    emb_pk = emb.reshape(n // 4, 128)

    wt = (W * scale).T
    b8 = (b * scale).reshape(d_model, 1)
    out3 = _proj(emb_pk, wt, b8, hist, batch)
    return jnp.transpose(out3, (2, 0, 1))


# speedup vs baseline: 1.2914x; 1.0438x over previous
import functools
import math

import jax
import jax.numpy as jnp
from jax import lax
from jax.experimental import pallas as pl
from jax.experimental.pallas import tpu as pltpu
from jax.experimental.pallas import tpu_sc as plsc

_NC = 2
_NS = 16
_NW = _NC * _NS

_CHUNK = 128
_GB = 8


def _gather_body(table_hbm, idx_hbm, out_hbm, idx_v, rows_v, sem):
    wid = lax.axis_index("s") * _NC + lax.axis_index("c")
    n_chunks = idx_hbm.shape[0]
    per_w = n_chunks // _NW

    def step(i, carry):
        base = wid * per_w + i * _GB
        pltpu.sync_copy(idx_hbm.at[pl.ds(base, _GB)], idx_v)
        copies = [
            pltpu.async_copy(table_hbm.at[idx_v.at[j]], rows_v.at[j], sem)
            for j in range(_GB)
        ]
        for c in copies:
            c.wait()
        pltpu.sync_copy(rows_v, out_hbm.at[pl.ds(base, _GB)])
        return carry

    lax.fori_loop(0, per_w // _GB, step, 0)


def _make_gather(n_chunks, embed):
    mesh = plsc.VectorSubcoreMesh(
        core_axis_name="c", subcore_axis_name="s",
        num_cores=_NC, num_subcores=_NS,
    )
    return pl.kernel(
        _gather_body,
        out_type=jax.ShapeDtypeStruct((n_chunks, _CHUNK, embed), jnp.float32),
        mesh=mesh,
        scratch_types=[
            pltpu.VMEM((_GB, _CHUNK), jnp.int32),
            pltpu.VMEM((_GB, _CHUNK, embed), jnp.float32),
            pltpu.SemaphoreType.DMA,
        ],
        compiler_params=pltpu.CompilerParams(use_tc_tiling_on_sc=False),
    )


_LB = 4


def _proj_body(e_ref, wt_ref, b_ref, out_ref):
    rows_per_l = e_ref.shape[0] // _LB
    for m in range(_LB):
        e = e_ref[pl.ds(m * rows_per_l, rows_per_l), :]
        parts = []
        for k in range(4):
            ek = e[:, 32 * k:32 * (k + 1)]
            ok = lax.dot_general(
                wt_ref[...], ek,
                dimension_numbers=(((1,), (1,)), ((), ())),
                preferred_element_type=jnp.float32,
            )
            parts.append(ok)
        out_ref[m] = jnp.concatenate(parts, axis=1) + b_ref[...]


def _proj(emb_pk, wt, b8, hist, batch):
    rows = emb_pk.shape[0]
    rows_per_l = rows // hist
    d = wt.shape[0]
    return pl.pallas_call(
        _proj_body,
        grid=(hist // _LB,),
        in_specs=[
            pl.BlockSpec((_LB * rows_per_l, 128), lambda i: (i, 0)),
            pl.BlockSpec((d, 32), lambda i: (0, 0)),
            pl.BlockSpec((d, 1), lambda i: (0, 0)),
        ],
        out_specs=pl.BlockSpec((_LB, d, batch), lambda i: (i, 0, 0)),
        out_shape=jax.ShapeDtypeStruct((hist, d, batch), jnp.float32),
        compiler_params=pltpu.CompilerParams(
            dimension_semantics=("arbitrary",),
        ),
    )(emb_pk, wt, b8)


def kernel(x, table, W, b):
    batch, hist = x.shape
    n = batch * hist
    embed = table.shape[1]
    d_model = W.shape[1]
    scale = math.sqrt(float(d_model))
    quarter = batch // 4

    xp = x.T.reshape(hist, 4, quarter).transpose(0, 2, 1)
    idx = xp.reshape(n // _CHUNK, _CHUNK).astype(jnp.int32)
    emb = _make_gather(n // _CHUNK, embed)(table, idx)

    emb_pk = emb.reshape(n // 4, 128)

    wt = (W * scale).T
    b8 = (b * scale).reshape(d_model, 1)
    out3 = _proj(emb_pk, wt, b8, hist, batch)
    return jnp.transpose(out3, (2, 0, 1))


# speedup vs baseline: 1.3128x; 1.0166x over previous
import functools
import math

import jax
import jax.numpy as jnp
from jax import lax
from jax.experimental import pallas as pl
from jax.experimental.pallas import tpu as pltpu
from jax.experimental.pallas import tpu_sc as plsc

_NC = 2
_NS = 16
_NW = _NC * _NS

_CHUNK = 128
_GB = 8


def _gather_body(table_hbm, idx_hbm, out_hbm, idx_v, rows_v, sem):
    wid = lax.axis_index("s") * _NC + lax.axis_index("c")
    n_chunks = idx_hbm.shape[0]
    per_w = n_chunks // _NW

    def step(i, carry):
        base = wid * per_w + i * _GB
        pltpu.sync_copy(idx_hbm.at[pl.ds(base, _GB)], idx_v)
        copies = [
            pltpu.async_copy(table_hbm.at[idx_v.at[j]], rows_v.at[j], sem)
            for j in range(_GB)
        ]
        for c in copies:
            c.wait()
        pltpu.sync_copy(rows_v, out_hbm.at[pl.ds(base, _GB)])
        return carry

    lax.fori_loop(0, per_w // _GB, step, 0)


def _make_gather(n_chunks, embed):
    mesh = plsc.VectorSubcoreMesh(
        core_axis_name="c", subcore_axis_name="s",
        num_cores=_NC, num_subcores=_NS,
    )
    return pl.kernel(
        _gather_body,
        out_type=jax.ShapeDtypeStruct((n_chunks, _CHUNK, embed), jnp.float32),
        mesh=mesh,
        scratch_types=[
            pltpu.VMEM((_GB, _CHUNK), jnp.int32),
            pltpu.VMEM((_GB, _CHUNK, embed), jnp.float32),
            pltpu.SemaphoreType.DMA,
        ],
        compiler_params=pltpu.CompilerParams(use_tc_tiling_on_sc=False),
    )


_LB = 8


def _proj_body(e_ref, wt_ref, b_ref, out_ref):
    rows_per_l = e_ref.shape[0] // _LB
    for m in range(_LB):
        e = e_ref[pl.ds(m * rows_per_l, rows_per_l), :]
        parts = []
        for k in range(4):
            ek = e[:, 32 * k:32 * (k + 1)]
            ok = lax.dot_general(
                wt_ref[...], ek,
                dimension_numbers=(((1,), (1,)), ((), ())),
                preferred_element_type=jnp.float32,
            )
            parts.append(ok)
        out_ref[m] = jnp.concatenate(parts, axis=1) + b_ref[...]


def _proj(emb_pk, wt, b8, hist, batch):
    rows = emb_pk.shape[0]
    rows_per_l = rows // hist
    d = wt.shape[0]
    return pl.pallas_call(
        _proj_body,
        grid=(hist // _LB,),
        in_specs=[
            pl.BlockSpec((_LB * rows_per_l, 128), lambda i: (i, 0)),
            pl.BlockSpec((d, 32), lambda i: (0, 0)),
            pl.BlockSpec((d, 1), lambda i: (0, 0)),
        ],
        out_specs=pl.BlockSpec((_LB, d, batch), lambda i: (i, 0, 0)),
        out_shape=jax.ShapeDtypeStruct((hist, d, batch), jnp.float32),
        compiler_params=pltpu.CompilerParams(
            dimension_semantics=("arbitrary",),
        ),
    )(emb_pk, wt, b8)


def kernel(x, table, W, b):
    batch, hist = x.shape
    n = batch * hist
    embed = table.shape[1]
    d_model = W.shape[1]
    scale = math.sqrt(float(d_model))
    quarter = batch // 4

    xp = x.T.reshape(hist, 4, quarter).transpose(0, 2, 1)
    idx = xp.reshape(n // _CHUNK, _CHUNK).astype(jnp.int32)
    emb = _make_gather(n // _CHUNK, embed)(table, idx)

    emb_pk = emb.reshape(n // 4, 128)

    wt = (W * scale).T
    b8 = (b * scale).reshape(d_model, 1)
    out3 = _proj(emb_pk, wt, b8, hist, batch)
    return jnp.transpose(out3, (2, 0, 1))
